# baseline (speedup 1.0000x reference)
import functools

import jax
import jax.numpy as jnp
from jax import lax
from jax.experimental import pallas as pl
from jax.experimental.pallas import tpu as pltpu
from jax.experimental.pallas import tpu_sc as plsc

_MAX_LEN = 8192
_PE_DIM = 128
_D_MODEL = 1024
_BATCH = 4
_OUT_D = _D_MODEL + _PE_DIM

_NW = 32
_ROWS_PER_W = _BATCH * _MAX_LEN // _NW
_CHUNK = 32
_NCHUNK = _ROWS_PER_W // _CHUNK


@functools.partial(
    pl.kernel,
    mesh=plsc.VectorSubcoreMesh(core_axis_name="c", subcore_axis_name="s"),
    out_type=jax.ShapeDtypeStruct((_BATCH * _MAX_LEN, _OUT_D), jnp.float32),
    scratch_types=[
        pltpu.VMEM((_CHUNK, _D_MODEL), jnp.float32),
        pltpu.VMEM((_CHUNK, _D_MODEL), jnp.float32),
        pltpu.VMEM((_CHUNK, _PE_DIM), jnp.float32),
        pltpu.VMEM((_CHUNK, _PE_DIM), jnp.float32),
        pltpu.SemaphoreType.DMA,
        pltpu.SemaphoreType.DMA,
    ],
)
def _sc_concat(x_hbm, pe_hbm, out_hbm, bx0, bx1, bp0, bp1, sem_i, sem_o):
    wid = lax.axis_index("s") * 2 + lax.axis_index("c")
    base = wid * _ROWS_PER_W
    t0 = base % _MAX_LEN
    bx = (bx0, bx1)
    bp = (bp0, bp1)

    def _in_descs(i, slot):
        r = base + i * _CHUNK
        t = t0 + i * _CHUNK
        cx = pltpu.make_async_copy(x_hbm.at[pl.ds(r, _CHUNK), :], bx[slot],
                                   sem_i)
        cp = pltpu.make_async_copy(pe_hbm.at[pl.ds(t, _CHUNK), :], bp[slot],
                                   sem_i)
        return cx, cp

    def _out_descs(i, slot):
        r = base + i * _CHUNK
        cx = pltpu.make_async_copy(
            bx[slot], out_hbm.at[pl.ds(r, _CHUNK), pl.ds(0, _D_MODEL)], sem_o)
        cp = pltpu.make_async_copy(
            bp[slot], out_hbm.at[pl.ds(r, _CHUNK), pl.ds(_D_MODEL, _PE_DIM)],
            sem_o)
        return cx, cp

    def _start(descs):
        for c in descs:
            c.start()

    def _wait(descs):
        for c in descs:
            c.wait()

    _start(_in_descs(0, 0))

    def _step(g):
        for s in range(2):
            i = g + s
            _wait(_in_descs(i, s))
            outs = _out_descs(i, s)
            _start(outs)

            @pl.when(i + 1 < _NCHUNK)
            def _():
                _start(_in_descs(i + 1, 1 - s))

            _wait(outs)

    pl.loop(0, _NCHUNK, step=2)(_step)


def kernel(x, pe_table):
    batch, max_len, d_model = x.shape
    x2 = x.reshape(batch * max_len, d_model)
    out = _sc_concat(x2, pe_table)
    return out.reshape(batch, max_len, _OUT_D)


# speedup vs baseline: 1.3504x; 1.3504x over previous
import jax
import jax.numpy as jnp
from jax.experimental import pallas as pl
from jax.experimental.pallas import tpu as pltpu

_MAX_LEN = 8192
_PE_DIM = 128
_D_MODEL = 1024
_BATCH = 4
_OUT_D = _D_MODEL + _PE_DIM
_S = 2048
_N = _BATCH * _MAX_LEN // _S
_NSLOT = 4


def _body(x_hbm, pe_hbm, out_hbm, buf, sem_in, sem_out):
    i = pl.program_id(0)

    def in_descs(j):
        slot = j % _NSLOT
        t0 = (j % (_MAX_LEN // _S)) * _S
        cx = pltpu.make_async_copy(
            x_hbm.at[pl.ds(j * _S, _S), :],
            buf.at[slot, :, pl.ds(0, _D_MODEL)],
            sem_in.at[slot])
        cp = pltpu.make_async_copy(
            pe_hbm.at[pl.ds(t0, _S), :],
            buf.at[slot, :, pl.ds(_D_MODEL, _PE_DIM)],
            sem_in.at[slot])
        return cx, cp

    def out_desc(j):
        slot = j % _NSLOT
        return pltpu.make_async_copy(
            buf.at[slot], out_hbm.at[pl.ds(j * _S, _S), :], sem_out.at[slot])

    @pl.when(i == 0)
    def _():
        for c in in_descs(0):
            c.start()
        for c in in_descs(1):
            c.start()

    @pl.when(i >= 2)
    def _():
        out_desc(i - 2).wait()

    @pl.when(i + 2 < _N)
    def _():
        for c in in_descs(i + 2):
            c.start()

    for c in in_descs(i):
        c.wait()
    out_desc(i).start()

    @pl.when(i == _N - 1)
    def _():
        out_desc(_N - 2).wait()
        out_desc(_N - 1).wait()


def kernel(x, pe_table):
    batch, max_len, d_model = x.shape
    x2 = x.reshape(batch * max_len, d_model)
    out = pl.pallas_call(
        _body,
        grid=(_N,),
        in_specs=[
            pl.BlockSpec(memory_space=pl.ANY),
            pl.BlockSpec(memory_space=pl.ANY),
        ],
        out_specs=pl.BlockSpec(memory_space=pl.ANY),
        out_shape=jax.ShapeDtypeStruct((batch * max_len, _OUT_D), jnp.float32),
        scratch_shapes=[
            pltpu.VMEM((_NSLOT, _S, _OUT_D), jnp.float32),
            pltpu.SemaphoreType.DMA((_NSLOT,)),
            pltpu.SemaphoreType.DMA((_NSLOT,)),
        ],
    )(x2, pe_table)
    return out.reshape(batch, max_len, _OUT_D)
